# baseline (speedup 1.0000x reference)
import functools

import jax
import jax.numpy as jnp
from jax import lax
from jax.experimental import pallas as pl
from jax.experimental.pallas import tpu as pltpu
from jax.experimental.pallas import tpu_sc as plsc

N = 10000
E = 320000
F_IN = 128
HID = 16
NUM_CLASSES = 40

NC = 2
NS = 16
NW = NC * NS
CHUNK = 128
NCHUNK = 80
EPW = NCHUNK * CHUNK
EPAD = NW * EPW
NPAD = 10240
STRIPE = NPAD // NS
SCRATCH = NPAD - 1
HBLK = 320


def _stage_a_body(x_ref, wcat_ref, wroot_ref, b_ref, t1_ref, r1_ref):
    xb = x_ref[...]
    t1_ref[...] = jnp.dot(xb, wcat_ref[...], preferred_element_type=jnp.float32)
    r1_ref[...] = (
        jnp.dot(xb, wroot_ref[...], preferred_element_type=jnp.float32)
        + b_ref[...]
    )


def _stage_a(x, w1cat, w1root, b1):
    return pl.pallas_call(
        _stage_a_body,
        grid=(10,),
        in_specs=[
            pl.BlockSpec((N // 10, F_IN), lambda b: (b, 0)),
            pl.BlockSpec((F_IN, 2 * HID), lambda b: (0, 0)),
            pl.BlockSpec((F_IN, HID), lambda b: (0, 0)),
            pl.BlockSpec((1, HID), lambda b: (0, 0)),
        ],
        out_specs=(
            pl.BlockSpec((N // 10, 2 * HID), lambda b: (b, 0)),
            pl.BlockSpec((N // 10, HID), lambda b: (b, 0)),
        ),
        out_shape=(
            jax.ShapeDtypeStruct((N, 2 * HID), jnp.float32),
            jax.ShapeDtypeStruct((N, HID), jnp.float32),
        ),
    )(x, w1cat, w1root, b1)


TAIL = E - (NW - 1) * EPW


def _stage_edges(ei_hbm, u_hbm, src_v, dst_v, u_v, wid, base):
    @pl.when(wid < NW - 1)
    def _():
        pltpu.sync_copy(ei_hbm.at[0, pl.ds(base, EPW)], src_v)
        pltpu.sync_copy(ei_hbm.at[1, pl.ds(base, EPW)], dst_v)
        pltpu.sync_copy(u_hbm.at[0, pl.ds(base, EPW)], u_v)

    @pl.when(wid == NW - 1)
    def _():
        pltpu.sync_copy(ei_hbm.at[0, pl.ds(base, TAIL)],
                        src_v.at[pl.ds(0, TAIL)])
        pltpu.sync_copy(ei_hbm.at[1, pl.ds(base, TAIL)],
                        dst_v.at[pl.ds(0, TAIL)])
        pltpu.sync_copy(u_hbm.at[0, pl.ds(base, TAIL)],
                        u_v.at[pl.ds(0, TAIL)])
        scr = jnp.full((16,), SCRATCH, jnp.int32)
        zz = jnp.zeros((16,), jnp.float32)

        def fill(i, _):
            off = TAIL + i * 16
            src_v[pl.ds(off, 16)] = scr
            dst_v[pl.ds(off, 16)] = scr
            u_v[pl.ds(off, 16)] = zz
            return 0

        lax.fori_loop(0, (EPW - TAIL) // 16, fill, 0)


def _edge_loop(compute_chunk, tbl_s, acc_s, src_v, dst_v, u_v, dstb, rows,
               msg, semg, sems):
    for b in range(2):
        pltpu.async_copy(tbl_s.at[src_v.at[pl.ds(b * CHUNK, CHUNK)]], rows[b],
                         semg[b])

    def loop_body(j2, _):
        for b in range(2):
            j = j2 * 2 + b
            @pl.when(j2 > 0)
            def _():
                pltpu.make_async_copy(msg[b], acc_s.at[dstb[b]],
                                      sems[b]).wait()
            for g in range(CHUNK // 16):
                dstb[b][pl.ds(g * 16, 16)] = (
                    dst_v[pl.ds(j * CHUNK + g * 16, 16)]
                )
            pltpu.make_async_copy(tbl_s.at[src_v.at[pl.ds(j * CHUNK, CHUNK)]],
                                  rows[b], semg[b]).wait()
            compute_chunk(rows[b], msg[b], u_v, j)
            @pl.when(j2 < NCHUNK // 2 - 1)
            def _():
                pltpu.async_copy(
                    tbl_s.at[src_v.at[pl.ds((j + 2) * CHUNK, CHUNK)]],
                    rows[b], semg[b])
            pltpu.async_copy(msg[b], acc_s.at[dstb[b]], sems[b], add=True)
        return 0

    lax.fori_loop(0, NCHUNK // 2, loop_body, 0)
    for b in range(2):
        pltpu.make_async_copy(msg[b], acc_s.at[dstb[b]], sems[b]).wait()


def _flush_acc(acc_s, out_hbm, c, s):
    pltpu.sync_copy(
        acc_s.at[pl.ds(s * STRIPE, STRIPE)],
        out_hbm.at[pl.ds(c * NPAD + s * STRIPE, STRIPE)],
    )


def _l1_compute(rows_v, msg_v, u_v, j):
    for g in range(CHUNK // 16):
        u16 = u_v[pl.ds(j * CHUNK + g * 16, 16)]
        for i in range(16):
            e = g * 16 + i
            u = u16[i]
            r0 = rows_v[e, pl.ds(0, 16)]
            d = rows_v[e, pl.ds(16, 16)]
            msg_v[e, pl.ds(0, 16)] = r0 + u * d


def _l2_compute(rows_v, msg_v, u_v, j):
    for g in range(CHUNK // 16):
        u16 = u_v[pl.ds(j * CHUNK + g * 16, 16)]
        for i in range(16):
            e = g * 16 + i
            u = u16[i]
            hr = rows_v[e, pl.ds(0, 16)]
            msg_v[e, pl.ds(0, 16)] = hr
            msg_v[e, pl.ds(16, 16)] = u * hr


def _sc_l1_body(t1_hbm, ei_hbm, u_hbm, zeros_hbm, out_hbm,
                acc_s, tbl_s, src_v, dst_v, u_v,
                dstb0, dstb1, rows0, rows1, msg0, msg1,
                semg0, semg1, sems0, sems1):
    c = lax.axis_index("c")
    s = lax.axis_index("s")
    wid = s * NC + c

    _stage_edges(ei_hbm, u_hbm, src_v, dst_v, u_v, wid, wid * EPW)

    @pl.when(s < NS - 1)
    def _():
        pltpu.sync_copy(t1_hbm.at[pl.ds(s * STRIPE, STRIPE)],
                        tbl_s.at[pl.ds(s * STRIPE, STRIPE)])

    @pl.when(s == NS - 1)
    def _():
        pltpu.sync_copy(t1_hbm.at[pl.ds((NS - 1) * STRIPE, N - (NS - 1) * STRIPE)],
                        tbl_s.at[pl.ds((NS - 1) * STRIPE, N - (NS - 1) * STRIPE)])
        pltpu.sync_copy(zeros_hbm.at[pl.ds(0, NPAD - N)],
                        tbl_s.at[pl.ds(N, NPAD - N)])

    pltpu.sync_copy(zeros_hbm, acc_s.at[pl.ds(s * STRIPE, STRIPE)])
    plsc.subcore_barrier()

    ones = jnp.ones((16,), jnp.float32)
    for m in (msg0, msg1):
        for e in range(CHUNK):
            m[e, pl.ds(HID, 16)] = ones

    _edge_loop(_l1_compute, tbl_s, acc_s, src_v, dst_v, u_v,
               (dstb0, dstb1), (rows0, rows1), (msg0, msg1),
               (semg0, semg1), (sems0, sems1))
    plsc.subcore_barrier()
    _flush_acc(acc_s, out_hbm, c, s)


def _sc_l1(t1p, eip, up, zeros):
    mesh = plsc.VectorSubcoreMesh(
        core_axis_name="c", subcore_axis_name="s", num_cores=NC, num_subcores=NS
    )
    return pl.kernel(
        _sc_l1_body,
        out_type=jax.ShapeDtypeStruct((NC * NPAD, 32), jnp.float32),
        mesh=mesh,
        compiler_params=pltpu.CompilerParams(use_tc_tiling_on_sc=False),
        scratch_types=[
            pltpu.VMEM_SHARED((NPAD, 32), jnp.float32),
            pltpu.VMEM_SHARED((NPAD, 2 * HID), jnp.float32),
            pltpu.VMEM((EPW,), jnp.int32),
            pltpu.VMEM((EPW,), jnp.int32),
            pltpu.VMEM((EPW,), jnp.float32),
            pltpu.VMEM((CHUNK,), jnp.int32),
            pltpu.VMEM((CHUNK,), jnp.int32),
            pltpu.VMEM((CHUNK, 2 * HID), jnp.float32),
            pltpu.VMEM((CHUNK, 2 * HID), jnp.float32),
            pltpu.VMEM((CHUNK, 32), jnp.float32),
            pltpu.VMEM((CHUNK, 32), jnp.float32),
            pltpu.SemaphoreType.DMA,
            pltpu.SemaphoreType.DMA,
            pltpu.SemaphoreType.DMA,
            pltpu.SemaphoreType.DMA,
        ],
        name="sc_edge_l1",
    )(t1p, eip, up, zeros)


def _sc_l2_body(p1_hbm, r1_hbm, ei_hbm, u_hbm, zeros_hbm,
                out_hbm, h_hbm, deg_hbm,
                acc_s, tbl_s, src_v, dst_v, u_v,
                dstb0, dstb1, rows0, rows1, msg0, msg1,
                pa_v, pb_v, r1_v, h_v, deg_v,
                semg0, semg1, sems0, sems1):
    c = lax.axis_index("c")
    s = lax.axis_index("s")
    wid = s * NC + c

    _stage_edges(ei_hbm, u_hbm, src_v, dst_v, u_v, wid, wid * EPW)
    pltpu.sync_copy(zeros_hbm, acc_s.at[pl.ds(s * STRIPE, STRIPE)])

    for blk in range(STRIPE // HBLK):
        row0 = s * STRIPE + blk * HBLK
        pltpu.sync_copy(p1_hbm.at[pl.ds(row0, HBLK)], pa_v)
        pltpu.sync_copy(p1_hbm.at[pl.ds(NPAD + row0, HBLK)], pb_v)
        pltpu.sync_copy(r1_hbm.at[pl.ds(row0, HBLK)], r1_v)

        def row_body(it, _):
            for k in range(4):
                r = it * 4 + k
                s0 = pa_v[r, pl.ds(0, HID)] + pb_v[r, pl.ds(0, HID)]
                dv = pa_v[r, pl.ds(HID, 16)] + pb_v[r, pl.ds(HID, 16)]
                dc = jnp.maximum(dv, 1.0)
                pre = s0 / dc + r1_v[r, pl.ds(0, HID)]
                h_v[r, pl.ds(0, HID)] = jnp.where(
                    pre > 0, pre, jnp.exp(jnp.minimum(pre, 0.0)) - 1.0)
                deg_v[r, pl.ds(0, HID)] = dc
            return 0

        lax.fori_loop(0, HBLK // 4, row_body, 0)
        pltpu.sync_copy(h_v, tbl_s.at[pl.ds(row0, HBLK)])
        pltpu.sync_copy(h_v, h_hbm.at[pl.ds(row0, HBLK)])
        pltpu.sync_copy(deg_v, deg_hbm.at[pl.ds(row0, HBLK)])

    plsc.subcore_barrier()

    _edge_loop(_l2_compute, tbl_s, acc_s, src_v, dst_v, u_v,
               (dstb0, dstb1), (rows0, rows1), (msg0, msg1),
               (semg0, semg1), (sems0, sems1))
    plsc.subcore_barrier()
    _flush_acc(acc_s, out_hbm, c, s)


def _sc_l2(p1, r1p, eip, up, zeros):
    mesh = plsc.VectorSubcoreMesh(
        core_axis_name="c", subcore_axis_name="s", num_cores=NC, num_subcores=NS
    )
    return pl.kernel(
        _sc_l2_body,
        out_type=(
            jax.ShapeDtypeStruct((NC * NPAD, 32), jnp.float32),
            jax.ShapeDtypeStruct((NPAD, HID), jnp.float32),
            jax.ShapeDtypeStruct((NPAD, HID), jnp.float32),
        ),
        mesh=mesh,
        compiler_params=pltpu.CompilerParams(use_tc_tiling_on_sc=False),
        scratch_types=[
            pltpu.VMEM_SHARED((NPAD, 32), jnp.float32),
            pltpu.VMEM_SHARED((NPAD, HID), jnp.float32),
            pltpu.VMEM((EPW,), jnp.int32),
            pltpu.VMEM((EPW,), jnp.int32),
            pltpu.VMEM((EPW,), jnp.float32),
            pltpu.VMEM((CHUNK,), jnp.int32),
            pltpu.VMEM((CHUNK,), jnp.int32),
            pltpu.VMEM((CHUNK, HID), jnp.float32),
            pltpu.VMEM((CHUNK, HID), jnp.float32),
            pltpu.VMEM((CHUNK, 32), jnp.float32),
            pltpu.VMEM((CHUNK, 32), jnp.float32),
            pltpu.VMEM((HBLK, 32), jnp.float32),
            pltpu.VMEM((HBLK, 32), jnp.float32),
            pltpu.VMEM((HBLK, HID), jnp.float32),
            pltpu.VMEM((HBLK, HID), jnp.float32),
            pltpu.VMEM((HBLK, HID), jnp.float32),
            pltpu.SemaphoreType.DMA,
            pltpu.SemaphoreType.DMA,
            pltpu.SemaphoreType.DMA,
            pltpu.SemaphoreType.DMA,
        ],
        name="sc_edge_l2",
    )(p1, r1p, eip, up, zeros)


def _stage_c_body(p2_ref, deg_ref, h_ref, w2cat_ref, w2root_ref, b_ref,
                  out_ref):
    s2 = p2_ref[pl.ds(0, N), :] + p2_ref[pl.ds(NPAD, N), :]
    agg = jnp.dot(s2, w2cat_ref[...], preferred_element_type=jnp.float32)
    agg = agg / deg_ref[pl.ds(0, N), :1]
    o = (
        agg
        + jnp.dot(h_ref[pl.ds(0, N), :], w2root_ref[...],
                  preferred_element_type=jnp.float32)
        + b_ref[...]
    )
    m = jnp.max(o, axis=1, keepdims=True)
    z = o - m
    lse = jnp.log(jnp.sum(jnp.exp(z), axis=1, keepdims=True))
    out_ref[...] = z - lse


def _stage_c(p2, deg, h, w2cat, w2root, b2):
    return pl.pallas_call(
        _stage_c_body,
        out_shape=jax.ShapeDtypeStruct((N, NUM_CLASSES), jnp.float32),
    )(p2, deg, h, w2cat, w2root, b2)


@jax.jit
def kernel(x, edge_index, edge_attr, W1, W1_root, b1, W2, W2_root, b2):
    eip = edge_index
    up = edge_attr.reshape(1, E)
    zeros = jnp.zeros((STRIPE, 32), jnp.float32)

    w1cat = jnp.concatenate([W1[0], W1[1] - W1[0]], axis=1)
    w2cat = jnp.concatenate([W2[0], W2[1] - W2[0]], axis=0)

    t1, r1 = _stage_a(x, w1cat, W1_root, b1.reshape(1, HID))
    r1p = jnp.pad(r1, ((0, NPAD - N), (0, 0)))
    p1 = _sc_l1(t1, eip, up, zeros)
    p2, h, deg = _sc_l2(p1, r1p, eip, up, zeros)
    out = _stage_c(p2, deg, h, w2cat, W2_root, b2.reshape(1, NUM_CLASSES))
    return out


# speedup vs baseline: 1.0608x; 1.0608x over previous
import functools

import jax
import jax.numpy as jnp
from jax import lax
from jax.experimental import pallas as pl
from jax.experimental.pallas import tpu as pltpu
from jax.experimental.pallas import tpu_sc as plsc

N = 10000
E = 320000
F_IN = 128
HID = 16
NUM_CLASSES = 40

NC = 2
NS = 16
NW = NC * NS
CHUNK = 128
NCHUNK = 80
EPW = NCHUNK * CHUNK
EPAD = NW * EPW
NPAD = 10240
STRIPE = NPAD // NS
SCRATCH = NPAD - 1
HBLK = 320


def _stage_a_body(x_ref, wcat_ref, wroot_ref, b_ref, t1_ref, r1_ref):
    xb = x_ref[...]
    t1_ref[...] = jnp.dot(xb, wcat_ref[...], preferred_element_type=jnp.float32)
    r1_ref[...] = (
        jnp.dot(xb, wroot_ref[...], preferred_element_type=jnp.float32)
        + b_ref[...]
    )


def _stage_a(x, w1cat, w1root, b1):
    return pl.pallas_call(
        _stage_a_body,
        grid=(10,),
        in_specs=[
            pl.BlockSpec((N // 10, F_IN), lambda b: (b, 0)),
            pl.BlockSpec((F_IN, 2 * HID), lambda b: (0, 0)),
            pl.BlockSpec((F_IN, HID), lambda b: (0, 0)),
            pl.BlockSpec((1, HID), lambda b: (0, 0)),
        ],
        out_specs=(
            pl.BlockSpec((N // 10, 2 * HID), lambda b: (b, 0)),
            pl.BlockSpec((N // 10, HID), lambda b: (b, 0)),
        ),
        out_shape=(
            jax.ShapeDtypeStruct((N, 2 * HID), jnp.float32),
            jax.ShapeDtypeStruct((N, HID), jnp.float32),
        ),
    )(x, w1cat, w1root, b1)


def _stage_edges(ei_hbm, u_hbm, src_v, dst_v, u_v, base):
    pltpu.sync_copy(ei_hbm.at[0, pl.ds(base, EPW)], src_v)
    pltpu.sync_copy(ei_hbm.at[1, pl.ds(base, EPW)], dst_v)
    pltpu.sync_copy(u_hbm.at[0, pl.ds(base, EPW)], u_v)


def _edge_loop(compute_chunk, tbl_s, acc_s, src_v, dst_v, u_v, dstb, rows,
               msg, semg, sems):
    for b in range(2):
        pltpu.async_copy(tbl_s.at[src_v.at[pl.ds(b * CHUNK, CHUNK)]], rows[b],
                         semg[b])

    def loop_body(j2, _):
        for b in range(2):
            j = j2 * 2 + b
            @pl.when(j2 > 0)
            def _():
                pltpu.make_async_copy(msg[b], acc_s.at[dstb[b]],
                                      sems[b]).wait()
            for g in range(CHUNK // 16):
                dstb[b][pl.ds(g * 16, 16)] = (
                    dst_v[pl.ds(j * CHUNK + g * 16, 16)]
                )
            pltpu.make_async_copy(tbl_s.at[src_v.at[pl.ds(j * CHUNK, CHUNK)]],
                                  rows[b], semg[b]).wait()
            compute_chunk(rows[b], msg[b], u_v, j)
            @pl.when(j2 < NCHUNK // 2 - 1)
            def _():
                pltpu.async_copy(
                    tbl_s.at[src_v.at[pl.ds((j + 2) * CHUNK, CHUNK)]],
                    rows[b], semg[b])
            pltpu.async_copy(msg[b], acc_s.at[dstb[b]], sems[b], add=True)
        return 0

    lax.fori_loop(0, NCHUNK // 2, loop_body, 0)
    for b in range(2):
        pltpu.make_async_copy(msg[b], acc_s.at[dstb[b]], sems[b]).wait()


def _flush_acc(acc_s, out_hbm, c, s):
    pltpu.sync_copy(
        acc_s.at[pl.ds(s * STRIPE, STRIPE)],
        out_hbm.at[pl.ds(c * NPAD + s * STRIPE, STRIPE)],
    )


def _l1_compute(rows_v, msg_v, u_v, j):
    for g in range(CHUNK // 16):
        u16 = u_v[pl.ds(j * CHUNK + g * 16, 16)]
        for i in range(16):
            e = g * 16 + i
            u = u16[i]
            r0 = rows_v[e, pl.ds(0, 16)]
            r1 = rows_v[e, pl.ds(16, 16)]
            msg_v[e, pl.ds(0, 16)] = r0 + u * (r1 - r0)


def _l2_compute(rows_v, msg_v, u_v, j):
    for g in range(CHUNK // 16):
        u16 = u_v[pl.ds(j * CHUNK + g * 16, 16)]
        for i in range(16):
            e = g * 16 + i
            u = u16[i]
            hr = rows_v[e, pl.ds(0, 16)]
            uh = u * hr
            msg_v[e, pl.ds(0, 16)] = hr - uh
            msg_v[e, pl.ds(16, 16)] = uh


def _sc_l1_body(t1_hbm, ei_hbm, u_hbm, zeros_hbm, out_hbm,
                acc_s, tbl_s, src_v, dst_v, u_v,
                dstb0, dstb1, rows0, rows1, msg0, msg1,
                semg0, semg1, sems0, sems1):
    c = lax.axis_index("c")
    s = lax.axis_index("s")
    wid = s * NC + c

    _stage_edges(ei_hbm, u_hbm, src_v, dst_v, u_v, wid * EPW)

    @pl.when(s < NS - 1)
    def _():
        pltpu.sync_copy(t1_hbm.at[pl.ds(s * STRIPE, STRIPE)],
                        tbl_s.at[pl.ds(s * STRIPE, STRIPE)])

    @pl.when(s == NS - 1)
    def _():
        pltpu.sync_copy(t1_hbm.at[pl.ds((NS - 1) * STRIPE, N - (NS - 1) * STRIPE)],
                        tbl_s.at[pl.ds((NS - 1) * STRIPE, N - (NS - 1) * STRIPE)])
        pltpu.sync_copy(zeros_hbm.at[pl.ds(0, NPAD - N)],
                        tbl_s.at[pl.ds(N, NPAD - N)])

    pltpu.sync_copy(zeros_hbm, acc_s.at[pl.ds(s * STRIPE, STRIPE)])
    plsc.subcore_barrier()

    ones = jnp.ones((16,), jnp.float32)
    for m in (msg0, msg1):
        for e in range(CHUNK):
            m[e, pl.ds(HID, 16)] = ones

    _edge_loop(_l1_compute, tbl_s, acc_s, src_v, dst_v, u_v,
               (dstb0, dstb1), (rows0, rows1), (msg0, msg1),
               (semg0, semg1), (sems0, sems1))
    plsc.subcore_barrier()
    _flush_acc(acc_s, out_hbm, c, s)


def _sc_l1(t1p, eip, up, zeros):
    mesh = plsc.VectorSubcoreMesh(
        core_axis_name="c", subcore_axis_name="s", num_cores=NC, num_subcores=NS
    )
    return pl.kernel(
        _sc_l1_body,
        out_type=jax.ShapeDtypeStruct((NC * NPAD, 32), jnp.float32),
        mesh=mesh,
        compiler_params=pltpu.CompilerParams(use_tc_tiling_on_sc=False),
        scratch_types=[
            pltpu.VMEM_SHARED((NPAD, 32), jnp.float32),
            pltpu.VMEM_SHARED((NPAD, 2 * HID), jnp.float32),
            pltpu.VMEM((EPW,), jnp.int32),
            pltpu.VMEM((EPW,), jnp.int32),
            pltpu.VMEM((EPW,), jnp.float32),
            pltpu.VMEM((CHUNK,), jnp.int32),
            pltpu.VMEM((CHUNK,), jnp.int32),
            pltpu.VMEM((CHUNK, 2 * HID), jnp.float32),
            pltpu.VMEM((CHUNK, 2 * HID), jnp.float32),
            pltpu.VMEM((CHUNK, 32), jnp.float32),
            pltpu.VMEM((CHUNK, 32), jnp.float32),
            pltpu.SemaphoreType.DMA,
            pltpu.SemaphoreType.DMA,
            pltpu.SemaphoreType.DMA,
            pltpu.SemaphoreType.DMA,
        ],
        name="sc_edge_l1",
    )(t1p, eip, up, zeros)


def _sc_l2_body(p1_hbm, r1_hbm, ei_hbm, u_hbm, zeros_hbm,
                out_hbm, h_hbm, deg_hbm,
                acc_s, tbl_s, src_v, dst_v, u_v,
                dstb0, dstb1, rows0, rows1, msg0, msg1,
                pa_v, pb_v, r1_v, h_v, deg_v,
                semg0, semg1, sems0, sems1):
    c = lax.axis_index("c")
    s = lax.axis_index("s")
    wid = s * NC + c

    _stage_edges(ei_hbm, u_hbm, src_v, dst_v, u_v, wid * EPW)
    pltpu.sync_copy(zeros_hbm, acc_s.at[pl.ds(s * STRIPE, STRIPE)])

    for blk in range(STRIPE // HBLK):
        row0 = s * STRIPE + blk * HBLK
        pltpu.sync_copy(p1_hbm.at[pl.ds(row0, HBLK)], pa_v)
        pltpu.sync_copy(p1_hbm.at[pl.ds(NPAD + row0, HBLK)], pb_v)
        pltpu.sync_copy(r1_hbm.at[pl.ds(row0, HBLK)], r1_v)

        def row_body(it, _):
            for k in range(4):
                r = it * 4 + k
                s0 = pa_v[r, pl.ds(0, HID)] + pb_v[r, pl.ds(0, HID)]
                dv = pa_v[r, pl.ds(HID, 16)] + pb_v[r, pl.ds(HID, 16)]
                dc = jnp.maximum(dv, 1.0)
                pre = s0 / dc + r1_v[r, pl.ds(0, HID)]
                h_v[r, pl.ds(0, HID)] = jnp.where(
                    pre > 0, pre, jnp.exp(jnp.minimum(pre, 0.0)) - 1.0)
                deg_v[r, pl.ds(0, HID)] = dc
            return 0

        lax.fori_loop(0, HBLK // 4, row_body, 0)
        pltpu.sync_copy(h_v, tbl_s.at[pl.ds(row0, HBLK)])
        pltpu.sync_copy(h_v, h_hbm.at[pl.ds(row0, HBLK)])
        pltpu.sync_copy(deg_v, deg_hbm.at[pl.ds(row0, HBLK)])

    plsc.subcore_barrier()

    _edge_loop(_l2_compute, tbl_s, acc_s, src_v, dst_v, u_v,
               (dstb0, dstb1), (rows0, rows1), (msg0, msg1),
               (semg0, semg1), (sems0, sems1))
    plsc.subcore_barrier()
    _flush_acc(acc_s, out_hbm, c, s)


def _sc_l2(p1, r1p, eip, up, zeros):
    mesh = plsc.VectorSubcoreMesh(
        core_axis_name="c", subcore_axis_name="s", num_cores=NC, num_subcores=NS
    )
    return pl.kernel(
        _sc_l2_body,
        out_type=(
            jax.ShapeDtypeStruct((NC * NPAD, 32), jnp.float32),
            jax.ShapeDtypeStruct((NPAD, HID), jnp.float32),
            jax.ShapeDtypeStruct((NPAD, HID), jnp.float32),
        ),
        mesh=mesh,
        compiler_params=pltpu.CompilerParams(use_tc_tiling_on_sc=False),
        scratch_types=[
            pltpu.VMEM_SHARED((NPAD, 32), jnp.float32),
            pltpu.VMEM_SHARED((NPAD, HID), jnp.float32),
            pltpu.VMEM((EPW,), jnp.int32),
            pltpu.VMEM((EPW,), jnp.int32),
            pltpu.VMEM((EPW,), jnp.float32),
            pltpu.VMEM((CHUNK,), jnp.int32),
            pltpu.VMEM((CHUNK,), jnp.int32),
            pltpu.VMEM((CHUNK, HID), jnp.float32),
            pltpu.VMEM((CHUNK, HID), jnp.float32),
            pltpu.VMEM((CHUNK, 32), jnp.float32),
            pltpu.VMEM((CHUNK, 32), jnp.float32),
            pltpu.VMEM((HBLK, 32), jnp.float32),
            pltpu.VMEM((HBLK, 32), jnp.float32),
            pltpu.VMEM((HBLK, HID), jnp.float32),
            pltpu.VMEM((HBLK, HID), jnp.float32),
            pltpu.VMEM((HBLK, HID), jnp.float32),
            pltpu.SemaphoreType.DMA,
            pltpu.SemaphoreType.DMA,
            pltpu.SemaphoreType.DMA,
            pltpu.SemaphoreType.DMA,
        ],
        name="sc_edge_l2",
    )(p1, r1p, eip, up, zeros)


def _stage_c_body(p2_ref, deg_ref, h_ref, w2cat_ref, w2root_ref, b_ref,
                  out_ref):
    s2 = p2_ref[pl.ds(0, N), :] + p2_ref[pl.ds(NPAD, N), :]
    agg = jnp.dot(s2, w2cat_ref[...], preferred_element_type=jnp.float32)
    agg = agg / deg_ref[pl.ds(0, N), :1]
    o = (
        agg
        + jnp.dot(h_ref[pl.ds(0, N), :], w2root_ref[...],
                  preferred_element_type=jnp.float32)
        + b_ref[...]
    )
    m = jnp.max(o, axis=1, keepdims=True)
    z = o - m
    lse = jnp.log(jnp.sum(jnp.exp(z), axis=1, keepdims=True))
    out_ref[...] = z - lse


def _stage_c(p2, deg, h, w2cat, w2root, b2):
    return pl.pallas_call(
        _stage_c_body,
        out_shape=jax.ShapeDtypeStruct((N, NUM_CLASSES), jnp.float32),
    )(p2, deg, h, w2cat, w2root, b2)


@jax.jit
def kernel(x, edge_index, edge_attr, W1, W1_root, b1, W2, W2_root, b2):
    eip = jnp.pad(edge_index, ((0, 0), (0, EPAD - E)),
                  constant_values=SCRATCH)
    up = jnp.pad(edge_attr, ((0, EPAD - E), (0, 0))).reshape(1, EPAD)
    zeros = jnp.zeros((STRIPE, 32), jnp.float32)

    w1cat = jnp.concatenate([W1[0], W1[1]], axis=1)
    w2cat = jnp.reshape(W2, (2 * HID, NUM_CLASSES))

    t1, r1 = _stage_a(x, w1cat, W1_root, b1.reshape(1, HID))
    r1p = jnp.pad(r1, ((0, NPAD - N), (0, 0)))
    p1 = _sc_l1(t1, eip, up, zeros)
    p2, h, deg = _sc_l2(p1, r1p, eip, up, zeros)
    out = _stage_c(p2, deg, h, w2cat, W2_root, b2.reshape(1, NUM_CLASSES))
    return out
